# baseline (speedup 1.0000x reference)
import dataclasses
import functools

import jax
import jax.numpy as jnp
from jax import lax
from jax.experimental import pallas as pl
from jax.experimental.pallas import tpu as pltpu
from jax.experimental.pallas import tpu_sc as plsc

L = 16
NW = 32

D_SP, D_IT, D_AB, D_MV, D_AT = 64, 32, 64, 128, 32
C_SP, C_IT, C_AB, C_MV, C_MA, C_PA = 0, 64, 96, 160, 672, 704
D_OUT = 736


def _build_sc_call(B):
    NB = B // NW
    assert B % (NW * L) == 0

    mesh = plsc.VectorSubcoreMesh(core_axis_name="c", subcore_axis_name="s")
    cp = pltpu.CompilerParams()
    fields_ = pltpu.CompilerParams.__dataclass_fields__
    if "needs_layout_passes" in fields_:
        cp = dataclasses.replace(cp, needs_layout_passes=False)
    if "use_tc_tiling_on_sc" in fields_:
        cp = dataclasses.replace(cp, use_tc_tiling_on_sc=False)

    @functools.partial(
        pl.kernel,
        out_type=[jax.ShapeDtypeStruct((B, D_OUT), jnp.float32)] * 12,
        mesh=mesh,
        compiler_params=cp,
        scratch_types=[
            pltpu.VMEM((NB * 12,), jnp.int32),
            pltpu.VMEM((NB * 12,), jnp.int32),
            pltpu.VMEM((NB * 12,), jnp.int32),
            pltpu.VMEM((NB * 48,), jnp.int32),
            pltpu.VMEM((NB,), jnp.int32),
            pltpu.VMEM((NB,), jnp.int32),
            pltpu.VMEM((NB,), jnp.int32),
            pltpu.VMEM((NB,), jnp.int32),
            pltpu.VMEM((NB,), jnp.int32),
            pltpu.VMEM((NB,), jnp.int32),
            pltpu.VMEM((NB,), jnp.int32),
            pltpu.VMEM((NB, D_SP), jnp.float32),
            pltpu.VMEM((NB, D_IT), jnp.float32),
            pltpu.VMEM((NB, D_AB), jnp.float32),
            pltpu.VMEM((NB, D_MV), jnp.float32),
            pltpu.VMEM((NB, D_MV), jnp.float32),
            pltpu.VMEM((NB, D_MV), jnp.float32),
            pltpu.VMEM((NB, D_MV), jnp.float32),
            pltpu.VMEM((NB, D_AT), jnp.float32),
            pltpu.VMEM((NB, D_AT), jnp.float32),
            pltpu.SemaphoreType.DMA,
            pltpu.SemaphoreType.DMA,
        ],
    )
    def sc_encoder(sp_hbm, mv_hbm, it_hbm, ab_hbm, ma_hbm, pa_hbm,
                   w_sp, w_mv, w_it, w_ab, *rest):
        outs = rest[:12]
        (sp_blk, it_blk, ab_blk, mv_blk,
         sp_idx, it_idx, ab_idx, mv_idx0, mv_idx1, mv_idx2, mv_idx3,
         sp_rows, it_rows, ab_rows, mv_rows0, mv_rows1, mv_rows2, mv_rows3,
         ma_buf, pa_buf, rsem, wsem) = rest[12:]
        mv_idx = (mv_idx0, mv_idx1, mv_idx2, mv_idx3)
        mv_rows = (mv_rows0, mv_rows1, mv_rows2, mv_rows3)

        wid = lax.axis_index("s") * 2 + lax.axis_index("c")
        b0 = wid * NB

        blk_loads = [
            pltpu.async_copy(sp_hbm.at[pl.ds(b0 * 12, NB * 12)], sp_blk, rsem),
            pltpu.async_copy(it_hbm.at[pl.ds(b0 * 12, NB * 12)], it_blk, rsem),
            pltpu.async_copy(ab_hbm.at[pl.ds(b0 * 12, NB * 12)], ab_blk, rsem),
            pltpu.async_copy(mv_hbm.at[pl.ds(b0 * 48, NB * 48)], mv_blk, rsem),
        ]
        for c in blk_loads:
            c.wait()

        iota = lax.iota(jnp.int32, L)
        i12 = iota * 12
        i48 = iota * 48

        def extract(blk, stride_iota, stride, col, dst):
            @pl.loop(0, NB // L)
            def _(v):
                rows = stride_iota + (v * (L * stride) + col)
                dst[pl.ds(v * L, L)] = plsc.load_gather(blk, [rows])

        for jj in range(12):
            extract(sp_blk, i12, 12, jj, sp_idx)
            extract(it_blk, i12, 12, jj, it_idx)
            extract(ab_blk, i12, 12, jj, ab_idx)
            for k in range(4):
                extract(mv_blk, i48, 48, jj * 4 + k, mv_idx[k])

            reads = [
                pltpu.async_copy(w_sp.at[sp_idx], sp_rows, rsem),
                pltpu.async_copy(w_it.at[it_idx], it_rows, rsem),
                pltpu.async_copy(w_ab.at[ab_idx], ab_rows, rsem),
            ]
            reads += [pltpu.async_copy(w_mv.at[mv_idx[k]], mv_rows[k], rsem)
                      for k in range(4)]
            reads += [
                pltpu.async_copy(ma_hbm.at[pl.ds(b0, NB), jj], ma_buf, rsem),
                pltpu.async_copy(pa_hbm.at[pl.ds(b0, NB), jj], pa_buf, rsem),
            ]
            for c in reads:
                c.wait()

            out = outs[jj]
            rows = pl.ds(b0, NB)
            writes = [
                pltpu.async_copy(sp_rows, out.at[rows, pl.ds(C_SP, D_SP)], wsem),
                pltpu.async_copy(it_rows, out.at[rows, pl.ds(C_IT, D_IT)], wsem),
                pltpu.async_copy(ab_rows, out.at[rows, pl.ds(C_AB, D_AB)], wsem),
            ]
            writes += [
                pltpu.async_copy(mv_rows[k],
                                 out.at[rows, pl.ds(C_MV + k * D_MV, D_MV)],
                                 wsem)
                for k in range(4)
            ]
            writes += [
                pltpu.async_copy(ma_buf, out.at[rows, pl.ds(C_MA, D_AT)], wsem),
                pltpu.async_copy(pa_buf, out.at[rows, pl.ds(C_PA, D_AT)], wsem),
            ]
            for c in writes:
                c.wait()

    return sc_encoder


def kernel(fields, sides, species, moves, items, abilities, move_attributes,
           pokemon_attributes, W_species, W_moves, W_items, W_abilities):
    B = fields.shape[0]
    sp = species.reshape(B * 12).astype(jnp.int32)
    mv = moves.reshape(B * 48).astype(jnp.int32)
    it = items.reshape(B * 12).astype(jnp.int32)
    ab = abilities.reshape(B * 12).astype(jnp.int32)
    ma = move_attributes.reshape(B, 12, 32)
    pa = pokemon_attributes.reshape(B, 12, 32)
    outs = _build_sc_call(B)(sp, mv, it, ab, ma, pa,
                             W_species, W_moves, W_items, W_abilities)
    one = fields[0, 0] * 0.0 + 1.0
    outs = [o * one for o in outs]
    pokemon_out = tuple(tuple(outs[j * 6 + i] for i in range(6))
                        for j in range(2))
    return (fields, sides, pokemon_out)


# speedup vs baseline: 1.2000x; 1.2000x over previous
import dataclasses
import functools

import jax
import jax.numpy as jnp
from jax import lax
from jax.experimental import pallas as pl
from jax.experimental.pallas import tpu as pltpu
from jax.experimental.pallas import tpu_sc as plsc

L = 16
NW = 32

D_SP, D_IT, D_AB, D_MV, D_AT = 64, 32, 64, 128, 32
C_SP, C_IT, C_AB, C_MV, C_MA, C_PA = 0, 64, 96, 160, 672, 704
D_OUT = 736


def _build_sc_call(B, jjs):
    NB = B // NW
    assert B % (NW * L) == 0

    mesh = plsc.VectorSubcoreMesh(core_axis_name="c", subcore_axis_name="s")
    cp = pltpu.CompilerParams()
    fields_ = pltpu.CompilerParams.__dataclass_fields__
    if "needs_layout_passes" in fields_:
        cp = dataclasses.replace(cp, needs_layout_passes=False)
    if "use_tc_tiling_on_sc" in fields_:
        cp = dataclasses.replace(cp, use_tc_tiling_on_sc=False)

    @functools.partial(
        pl.kernel,
        out_type=[jax.ShapeDtypeStruct((B, D_OUT), jnp.float32)] * len(jjs),
        mesh=mesh,
        compiler_params=cp,
        scratch_types=[
            pltpu.VMEM((NB * 12,), jnp.int32),
            pltpu.VMEM((NB * 12,), jnp.int32),
            pltpu.VMEM((NB * 12,), jnp.int32),
            pltpu.VMEM((NB * 48,), jnp.int32),
            pltpu.VMEM((NB,), jnp.int32),
            pltpu.VMEM((NB,), jnp.int32),
            pltpu.VMEM((NB,), jnp.int32),
            pltpu.VMEM((NB,), jnp.int32),
            pltpu.VMEM((NB,), jnp.int32),
            pltpu.VMEM((NB,), jnp.int32),
            pltpu.VMEM((NB,), jnp.int32),
            pltpu.VMEM((NB, D_SP), jnp.float32),
            pltpu.VMEM((NB, D_IT), jnp.float32),
            pltpu.VMEM((NB, D_AB), jnp.float32),
            pltpu.VMEM((NB, D_MV), jnp.float32),
            pltpu.VMEM((NB, D_MV), jnp.float32),
            pltpu.VMEM((NB, D_MV), jnp.float32),
            pltpu.VMEM((NB, D_MV), jnp.float32),
            pltpu.VMEM((NB, D_AT), jnp.float32),
            pltpu.VMEM((NB, D_AT), jnp.float32),
            pltpu.SemaphoreType.DMA,
            pltpu.SemaphoreType.DMA,
        ],
    )
    def sc_encoder(sp_hbm, mv_hbm, it_hbm, ab_hbm, ma_hbm, pa_hbm,
                   w_sp, w_mv, w_it, w_ab, *rest):
        outs = rest[:len(jjs)]
        (sp_blk, it_blk, ab_blk, mv_blk,
         sp_idx, it_idx, ab_idx, mv_idx0, mv_idx1, mv_idx2, mv_idx3,
         sp_rows, it_rows, ab_rows, mv_rows0, mv_rows1, mv_rows2, mv_rows3,
         ma_buf, pa_buf, rsem, wsem) = rest[len(jjs):]
        mv_idx = (mv_idx0, mv_idx1, mv_idx2, mv_idx3)
        mv_rows = (mv_rows0, mv_rows1, mv_rows2, mv_rows3)

        wid = lax.axis_index("s") * 2 + lax.axis_index("c")
        b0 = wid * NB

        blk_loads = [
            pltpu.async_copy(sp_hbm.at[pl.ds(b0 * 12, NB * 12)], sp_blk, rsem),
            pltpu.async_copy(it_hbm.at[pl.ds(b0 * 12, NB * 12)], it_blk, rsem),
            pltpu.async_copy(ab_hbm.at[pl.ds(b0 * 12, NB * 12)], ab_blk, rsem),
            pltpu.async_copy(mv_hbm.at[pl.ds(b0 * 48, NB * 48)], mv_blk, rsem),
        ]
        for c in blk_loads:
            c.wait()

        iota = lax.iota(jnp.int32, L)
        i12 = iota * 12
        i48 = iota * 48

        def extract(blk, stride_iota, stride, col, dst):
            @pl.loop(0, NB // L)
            def _(v):
                rows = stride_iota + (v * (L * stride) + col)
                dst[pl.ds(v * L, L)] = plsc.load_gather(blk, [rows])

        for oi, jj in enumerate(jjs):
            extract(sp_blk, i12, 12, jj, sp_idx)
            extract(it_blk, i12, 12, jj, it_idx)
            extract(ab_blk, i12, 12, jj, ab_idx)
            for k in range(4):
                extract(mv_blk, i48, 48, jj * 4 + k, mv_idx[k])

            reads = [
                pltpu.async_copy(w_sp.at[sp_idx], sp_rows, rsem),
                pltpu.async_copy(w_it.at[it_idx], it_rows, rsem),
                pltpu.async_copy(w_ab.at[ab_idx], ab_rows, rsem),
            ]
            reads += [pltpu.async_copy(w_mv.at[mv_idx[k]], mv_rows[k], rsem)
                      for k in range(4)]
            reads += [
                pltpu.async_copy(ma_hbm.at[pl.ds(b0, NB), jj], ma_buf, rsem),
                pltpu.async_copy(pa_hbm.at[pl.ds(b0, NB), jj], pa_buf, rsem),
            ]
            for c in reads:
                c.wait()

            out = outs[oi]
            rows = pl.ds(b0, NB)
            writes = [
                pltpu.async_copy(sp_rows, out.at[rows, pl.ds(C_SP, D_SP)], wsem),
                pltpu.async_copy(it_rows, out.at[rows, pl.ds(C_IT, D_IT)], wsem),
                pltpu.async_copy(ab_rows, out.at[rows, pl.ds(C_AB, D_AB)], wsem),
            ]
            writes += [
                pltpu.async_copy(mv_rows[k],
                                 out.at[rows, pl.ds(C_MV + k * D_MV, D_MV)],
                                 wsem)
                for k in range(4)
            ]
            writes += [
                pltpu.async_copy(ma_buf, out.at[rows, pl.ds(C_MA, D_AT)], wsem),
                pltpu.async_copy(pa_buf, out.at[rows, pl.ds(C_PA, D_AT)], wsem),
            ]
            for c in writes:
                c.wait()

    return sc_encoder


def kernel(fields, sides, species, moves, items, abilities, move_attributes,
           pokemon_attributes, W_species, W_moves, W_items, W_abilities):
    B = fields.shape[0]
    sp = species.reshape(B * 12).astype(jnp.int32)
    mv = moves.reshape(B * 48).astype(jnp.int32)
    it = items.reshape(B * 12).astype(jnp.int32)
    ab = abilities.reshape(B * 12).astype(jnp.int32)
    ma = move_attributes.reshape(B, 12, 32)
    pa = pokemon_attributes.reshape(B, 12, 32)
    jjs_a = tuple(range(6))
    jjs_b = tuple(range(6, 12))
    args = (sp, mv, it, ab, ma, pa,
            W_species, W_moves, W_items, W_abilities)
    outs_a = _build_sc_call(B, jjs_a)(*args)
    outs_b = _build_sc_call(B, jjs_b)(*args)
    one = fields[0, 0] * 0.0 + 1.0
    outs = [o * one for o in outs_a] + list(outs_b)
    pokemon_out = tuple(tuple(outs[j * 6 + i] for i in range(6))
                        for j in range(2))
    return (fields, sides, pokemon_out)


# speedup vs baseline: 1.3467x; 1.1222x over previous
import dataclasses
import functools

import jax
import jax.numpy as jnp
from jax import lax
from jax.experimental import pallas as pl
from jax.experimental.pallas import tpu as pltpu
from jax.experimental.pallas import tpu_sc as plsc

L = 16
NW = 32

D_SP, D_IT, D_AB, D_MV, D_AT = 64, 32, 64, 128, 32
C_SP, C_IT, C_AB, C_MV, C_MA, C_PA = 0, 64, 96, 160, 672, 704
D_OUT = 736


def _build_sc_call(B):
    NB = B // NW
    assert B % (NW * L) == 0

    mesh = plsc.VectorSubcoreMesh(core_axis_name="c", subcore_axis_name="s")
    cp = pltpu.CompilerParams()
    fields_ = pltpu.CompilerParams.__dataclass_fields__
    if "needs_layout_passes" in fields_:
        cp = dataclasses.replace(cp, needs_layout_passes=False)
    if "use_tc_tiling_on_sc" in fields_:
        cp = dataclasses.replace(cp, use_tc_tiling_on_sc=False)

    small_set = [
        pltpu.VMEM((NB, D_SP), jnp.float32),
        pltpu.VMEM((NB, D_IT), jnp.float32),
        pltpu.VMEM((NB, D_AB), jnp.float32),
        pltpu.VMEM((NB, D_AT), jnp.float32),
        pltpu.VMEM((NB, D_AT), jnp.float32),
    ]
    idx_set = [pltpu.VMEM((NB,), jnp.int32)] * 3

    @functools.partial(
        pl.kernel,
        out_type=[jax.ShapeDtypeStruct((B, D_OUT), jnp.float32)] * 12,
        mesh=mesh,
        compiler_params=cp,
        scratch_types=[
            pltpu.VMEM((NB * 12,), jnp.int32),
            pltpu.VMEM((NB * 12,), jnp.int32),
            pltpu.VMEM((NB * 12,), jnp.int32),
            pltpu.VMEM((NB * 48,), jnp.int32),
            *idx_set, *idx_set,
            pltpu.VMEM((NB,), jnp.int32),
            pltpu.VMEM((NB,), jnp.int32),
            *small_set, *small_set,
            pltpu.VMEM((NB, D_MV), jnp.float32),
            pltpu.VMEM((NB, D_MV), jnp.float32),
            pltpu.SemaphoreType.DMA,
            pltpu.SemaphoreType.DMA,
            pltpu.SemaphoreType.DMA,
            pltpu.SemaphoreType.DMA,
            pltpu.SemaphoreType.DMA,
            pltpu.SemaphoreType.DMA,
            pltpu.SemaphoreType.DMA,
        ],
    )
    def sc_encoder(sp_hbm, mv_hbm, it_hbm, ab_hbm, ma_hbm, pa_hbm,
                   w_sp, w_mv, w_it, w_ab, *rest):
        outs = rest[:12]
        (sp_blk, it_blk, ab_blk, mv_blk, *more) = rest[12:]
        idxsets = (more[0:3], more[3:6])
        mv_lists = (more[6], more[7])
        smallsets = (more[8:13], more[13:18])
        mv_bufs = (more[18], more[19])
        rsem = more[20]
        mv_rsems = (more[21], more[22])
        wsems = (more[23], more[24])
        mv_wsems = (more[25], more[26])

        wid = lax.axis_index("s") * 2 + lax.axis_index("c")
        b0 = wid * NB
        rows = pl.ds(b0, NB)

        blk_loads = [
            pltpu.async_copy(sp_hbm.at[pl.ds(b0 * 12, NB * 12)], sp_blk, rsem),
            pltpu.async_copy(it_hbm.at[pl.ds(b0 * 12, NB * 12)], it_blk, rsem),
            pltpu.async_copy(ab_hbm.at[pl.ds(b0 * 12, NB * 12)], ab_blk, rsem),
            pltpu.async_copy(mv_hbm.at[pl.ds(b0 * 48, NB * 48)], mv_blk, rsem),
        ]
        for c in blk_loads:
            c.wait()

        iota = lax.iota(jnp.int32, L)
        i12 = iota * 12
        i48 = iota * 48

        def extract(blk, stride_iota, stride, col, dst):
            @pl.loop(0, NB // L)
            def _(v):
                rws = stride_iota + (v * (L * stride) + col)
                dst[pl.ds(v * L, L)] = plsc.load_gather(blk, [rws])

        small_writes = [None, None]
        mv_write = [None, None]
        mv_prev = None

        def mv_step(jj, k):
            nonlocal mv_prev
            m = jj * 4 + k
            slot = m % 2
            if mv_write[slot] is not None:
                mv_write[slot].wait()
            extract(mv_blk, i48, 48, jj * 4 + k, mv_lists[slot])
            g = pltpu.async_copy(w_mv.at[mv_lists[slot]], mv_bufs[slot],
                                 mv_rsems[slot])
            if mv_prev is not None:
                pslot, pg, pout, pcol = mv_prev
                pg.wait()
                mv_write[pslot] = pltpu.async_copy(
                    mv_bufs[pslot], pout.at[rows, pl.ds(pcol, D_MV)],
                    mv_wsems[pslot])
            mv_prev = (slot, g, outs[jj], C_MV + k * D_MV)

        for jj in range(12):
            p = jj % 2
            sp_rows, it_rows, ab_rows, ma_buf, pa_buf = smallsets[p]
            sp_idx, it_idx, ab_idx = idxsets[p]
            if small_writes[p] is not None:
                for h in small_writes[p]:
                    h.wait()
            extract(sp_blk, i12, 12, jj, sp_idx)
            extract(it_blk, i12, 12, jj, it_idx)
            extract(ab_blk, i12, 12, jj, ab_idx)
            reads = [
                pltpu.async_copy(w_sp.at[sp_idx], sp_rows, rsem),
                pltpu.async_copy(w_it.at[it_idx], it_rows, rsem),
                pltpu.async_copy(w_ab.at[ab_idx], ab_rows, rsem),
                pltpu.async_copy(ma_hbm.at[pl.ds(b0, NB), jj], ma_buf, rsem),
                pltpu.async_copy(pa_hbm.at[pl.ds(b0, NB), jj], pa_buf, rsem),
            ]

            for k in range(4):
                mv_step(jj, k)

            for c in reads:
                c.wait()
            out = outs[jj]
            wsem = wsems[p]
            small_writes[p] = [
                pltpu.async_copy(sp_rows, out.at[rows, pl.ds(C_SP, D_SP)], wsem),
                pltpu.async_copy(it_rows, out.at[rows, pl.ds(C_IT, D_IT)], wsem),
                pltpu.async_copy(ab_rows, out.at[rows, pl.ds(C_AB, D_AB)], wsem),
                pltpu.async_copy(ma_buf, out.at[rows, pl.ds(C_MA, D_AT)], wsem),
                pltpu.async_copy(pa_buf, out.at[rows, pl.ds(C_PA, D_AT)], wsem),
            ]

        pslot, pg, pout, pcol = mv_prev
        pg.wait()
        mv_write[pslot] = pltpu.async_copy(
            mv_bufs[pslot], pout.at[rows, pl.ds(pcol, D_MV)], mv_wsems[pslot])
        for h in mv_write:
            if h is not None:
                h.wait()
        for hs in small_writes:
            if hs is not None:
                for h in hs:
                    h.wait()

    return sc_encoder


def kernel(fields, sides, species, moves, items, abilities, move_attributes,
           pokemon_attributes, W_species, W_moves, W_items, W_abilities):
    B = fields.shape[0]
    sp = species.reshape(B * 12).astype(jnp.int32)
    mv = moves.reshape(B * 48).astype(jnp.int32)
    it = items.reshape(B * 12).astype(jnp.int32)
    ab = abilities.reshape(B * 12).astype(jnp.int32)
    ma = move_attributes.reshape(B, 12, 32)
    pa = pokemon_attributes.reshape(B, 12, 32)
    outs = _build_sc_call(B)(sp, mv, it, ab, ma, pa,
                             W_species, W_moves, W_items, W_abilities)
    pokemon_out = tuple(tuple(outs[j * 6 + i] for i in range(6))
                        for j in range(2))
    return (fields, sides, pokemon_out)
